# baseline (speedup 1.0000x reference)
import functools

import jax
import jax.numpy as jnp
from jax import lax
from jax.experimental import pallas as pl
from jax.experimental.pallas import tpu as pltpu
from jax.experimental.pallas import tpu_sc as plsc

N = 10000
E = 320000
H = 128
HALF = H // 2

NC = 2
NS = 16
NW = NC * NS
EPW = E // NW
CHUNK = 80
NCHUNK = EPW // CHUNK
RPS = 624
TAIL = N - NS * RPS

_PREC = jax.lax.Precision.HIGHEST


def _dot(a, b):
    return jnp.dot(a, b, preferred_element_type=jnp.float32, precision=_PREC)


def _emb_node_body(nf_ref, x_ref, nw_ref, nb_ref, xe_ref, out_ref):
    nf = nf_ref[...]
    xv = x_ref[...]
    w = nw_ref[...]
    out_ref[...] = (nf[:, 0:1] * w[0:1, :] + nf[:, 1:2] * w[1:2, :]
                    + nb_ref[...]
                    + jnp.where(xv == 1, xe_ref[1:2, :], xe_ref[0:1, :]))


def _emb_nodes(nodes_feature, x2d, node_w, node_b, x_embed):
    nb = 2000
    grid = N // nb
    return pl.pallas_call(
        _emb_node_body,
        grid=(grid,),
        in_specs=[
            pl.BlockSpec((nb, 2), lambda i: (i, 0)),
            pl.BlockSpec((nb, 1), lambda i: (i, 0)),
            pl.BlockSpec((2, H), lambda i: (0, 0)),
            pl.BlockSpec((1, H), lambda i: (0, 0)),
            pl.BlockSpec((2, H), lambda i: (0, 0)),
        ],
        out_specs=pl.BlockSpec((nb, H), lambda i: (i, 0)),
        out_shape=jax.ShapeDtypeStruct((N, H), jnp.float32),
    )(nodes_feature, x2d, node_w, node_b, x_embed)


def _time_body(t_ref, w1_ref, b1_ref, w2_ref, b2_ref, etw_ref, etb_ref, out_ref):
    idx = jax.lax.broadcasted_iota(jnp.int32, (1, H), 1).astype(jnp.float32)
    k = jnp.where(idx < HALF, idx, idx - HALF)
    freq = jnp.exp((-jnp.log(10000.0) / HALF) * k)
    arg = t_ref[0, 0] * freq
    te = jnp.where(idx < HALF, jnp.sin(arg), jnp.cos(arg))
    h1 = jnp.maximum(_dot(te, w1_ref[...]) + b1_ref[...], 0.0)
    th = _dot(h1, w2_ref[...]) + b2_ref[...]
    rt = jnp.maximum(th, 0.0)
    out_ref[0:1, :] = _dot(rt, etw_ref[0]) + etb_ref[0:1, :]
    out_ref[1:2, :] = _dot(rt, etw_ref[1]) + etb_ref[1:2, :]


def _time_vecs(t11, w1, b1, w2, b2, etw, etb):
    return pl.pallas_call(
        _time_body,
        out_shape=jax.ShapeDtypeStruct((2, H), jnp.float32),
    )(t11, w1, b1, w2, b2, etw, etb)


def _mm_body(a_ref, w_ref, of_ref, ob_ref):
    prod = _dot(a_ref[...], w_ref[...])
    of_ref[...] = prod[:, 0:2 * H]
    ob_ref[...] = prod[:, 2 * H:4 * H].astype(jnp.bfloat16)


def _node_mats(x_h, w4):
    nb = 2000
    grid = N // nb
    return pl.pallas_call(
        _mm_body,
        grid=(grid,),
        in_specs=[
            pl.BlockSpec((nb, H), lambda i: (i, 0)),
            pl.BlockSpec((H, 4 * H), lambda i: (0, 0)),
        ],
        out_specs=[
            pl.BlockSpec((nb, 2 * H), lambda i: (i, 0)),
            pl.BlockSpec((nb, 2 * H), lambda i: (i, 0)),
        ],
        out_shape=[
            jax.ShapeDtypeStruct((N, 2 * H), jnp.float32),
            jax.ShapeDtypeStruct((N, 2 * H), jnp.bfloat16),
        ],
    )(x_h, w4)


def _edge_core(eh, sv_ref, ewg_ref, c_ref, g_ref, b_ref, tv_ref):
    sv = sv_ref[...]
    dx = sv[:, 0:H].astype(jnp.float32)
    vx = sv[:, H:2 * H].astype(jnp.float32)
    en = _dot(eh, c_ref[...]) + dx + ewg_ref[...]
    gates = 1.0 / (1.0 + jnp.exp(-en))
    msg = gates * vx
    mu = jnp.mean(en, axis=-1, keepdims=True)
    d = en - mu
    var = jnp.mean(d * d, axis=-1, keepdims=True)
    ln = g_ref[...] * d * jax.lax.rsqrt(var + 1e-5) + b_ref[...]
    ehn = eh + jnp.maximum(ln, 0.0) + tv_ref[...]
    return msg, ehn


def _ef_mid_body(eh_ref, sv_ref, ewg_ref, c_ref, g_ref, b_ref, tv_ref,
                 msg_ref, ehn_ref):
    msg_ref[...], ehn_ref[...] = _edge_core(
        eh_ref[...], sv_ref, ewg_ref, c_ref, g_ref, b_ref, tv_ref)


def _ef_first_body(ef_ref, e_ref, m_ref, ew0_ref, eb0_ref, ee_ref, me_ref,
                   sv_ref, ewg_ref, c_ref, g_ref, b_ref, tv_ref,
                   msg_ref, ehn_ref):
    eh = (ef_ref[...] * ew0_ref[...] + eb0_ref[...]
          + jnp.where(e_ref[...] == 1, ee_ref[1:2, :], ee_ref[0:1, :])
          + jnp.where(m_ref[...] == 1, me_ref[1:2, :], me_ref[0:1, :]))
    msg_ref[...], ehn_ref[...] = _edge_core(
        eh, sv_ref, ewg_ref, c_ref, g_ref, b_ref, tv_ref)


def _ef_last_body(eh_ref, sv_ref, ewg_ref, c_ref, g_ref, b_ref, tv_ref,
                  ow_ref, ob_ref, msg_ref, eo_ref):
    msg, ehn = _edge_core(
        eh_ref[...], sv_ref, ewg_ref, c_ref, g_ref, b_ref, tv_ref)
    msg_ref[...] = msg
    eo_ref[...] = _dot(ehn, ow_ref[...]) + ob_ref[...]


_EB = 4000
_COMMON_SPECS = [
    pl.BlockSpec((_EB, 2 * H), lambda i: (i, 0)),
    pl.BlockSpec((_EB, H), lambda i: (i, 0)),
    pl.BlockSpec((H, H), lambda i: (0, 0)),
    pl.BlockSpec((1, H), lambda i: (0, 0)),
    pl.BlockSpec((1, H), lambda i: (0, 0)),
    pl.BlockSpec((1, H), lambda i: (0, 0)),
]
_MSG_EHN_OUT = dict(
    out_specs=[
        pl.BlockSpec((_EB, H), lambda i: (i, 0)),
        pl.BlockSpec((_EB, H), lambda i: (i, 0)),
    ],
    out_shape=[
        jax.ShapeDtypeStruct((E, H), jnp.float32),
        jax.ShapeDtypeStruct((E, H), jnp.float32),
    ],
)


def _edge_fused_mid(e_h, sv, ewg, c, g, b, tvec):
    return pl.pallas_call(
        _ef_mid_body,
        grid=(E // _EB,),
        in_specs=[pl.BlockSpec((_EB, H), lambda i: (i, 0))] + _COMMON_SPECS,
        **_MSG_EHN_OUT,
    )(e_h, sv, ewg, c, g, b, tvec)


def _edge_fused_first(ef2d, e2d, m2d, ew0, eb0, ee, me, sv, ewg, c, g, b,
                      tvec):
    return pl.pallas_call(
        _ef_first_body,
        grid=(E // _EB,),
        in_specs=[
            pl.BlockSpec((_EB, 1), lambda i: (i, 0)),
            pl.BlockSpec((_EB, 1), lambda i: (i, 0)),
            pl.BlockSpec((_EB, 1), lambda i: (i, 0)),
            pl.BlockSpec((1, H), lambda i: (0, 0)),
            pl.BlockSpec((1, H), lambda i: (0, 0)),
            pl.BlockSpec((2, H), lambda i: (0, 0)),
            pl.BlockSpec((2, H), lambda i: (0, 0)),
        ] + _COMMON_SPECS,
        **_MSG_EHN_OUT,
    )(ef2d, e2d, m2d, ew0, eb0, ee, me, sv, ewg, c, g, b, tvec)


def _edge_fused_last(e_h, sv, ewg, c, g, b, tvec, ow, ob):
    return pl.pallas_call(
        _ef_last_body,
        grid=(E // _EB,),
        in_specs=[pl.BlockSpec((_EB, H), lambda i: (i, 0))] + _COMMON_SPECS
        + [
            pl.BlockSpec((H, 2), lambda i: (0, 0)),
            pl.BlockSpec((1, 2), lambda i: (0, 0)),
        ],
        out_specs=[
            pl.BlockSpec((_EB, H), lambda i: (i, 0)),
            pl.BlockSpec((_EB, 2), lambda i: (i, 0)),
        ],
        out_shape=[
            jax.ShapeDtypeStruct((E, H), jnp.float32),
            jax.ShapeDtypeStruct((E, 2), jnp.float32),
        ],
    )(e_h, sv, ewg, c, g, b, tvec, ow, ob)


def _node_upd_body(xh_ref, ux_ref, agg_ref, g_ref, b_ref, out_ref):
    xn = ux_ref[...] + agg_ref[0] + agg_ref[1]
    mu = jnp.mean(xn, axis=-1, keepdims=True)
    d = xn - mu
    var = jnp.mean(d * d, axis=-1, keepdims=True)
    ln = g_ref[...] * d * jax.lax.rsqrt(var + 1e-5) + b_ref[...]
    out_ref[...] = xh_ref[...] + jnp.maximum(ln, 0.0)


def _node_update(x_h, ux, agg, g, b):
    nb = 2000
    grid = N // nb
    return pl.pallas_call(
        _node_upd_body,
        grid=(grid,),
        in_specs=[
            pl.BlockSpec((nb, H), lambda i: (i, 0)),
            pl.BlockSpec((nb, H), lambda i: (i, 0)),
            pl.BlockSpec((NC, nb, H), lambda i: (0, i, 0)),
            pl.BlockSpec((1, H), lambda i: (0, 0)),
            pl.BlockSpec((1, H), lambda i: (0, 0)),
        ],
        out_specs=pl.BlockSpec((nb, H), lambda i: (i, 0)),
        out_shape=jax.ShapeDtypeStruct((N, H), jnp.float32),
    )(x_h, ux, agg, g, b)


def _proj_body(h_ref, w_ref, b_ref, out_ref):
    out_ref[...] = _dot(h_ref[...], w_ref[...]) + b_ref[...]


def _proj(h, w, b, rows, rb):
    grid = rows // rb
    return pl.pallas_call(
        _proj_body,
        grid=(grid,),
        in_specs=[
            pl.BlockSpec((rb, H), lambda i: (i, 0)),
            pl.BlockSpec((H, 2), lambda i: (0, 0)),
            pl.BlockSpec((1, 2), lambda i: (0, 0)),
        ],
        out_specs=pl.BlockSpec((rb, 2), lambda i: (i, 0)),
        out_shape=jax.ShapeDtypeStruct((rows, 2), jnp.float32),
    )(h, w, b)


def _sc_gather_body(svt_hbm, ewt_hbm, src_hbm, dst_hbm,
                    svg_out, ewg_out,
                    srcb, dstb, sgv, egv, gsem, wsem):
    c = lax.axis_index("c")
    s = lax.axis_index("s")
    wid = s * NC + c
    base = wid * EPW

    pltpu.sync_copy(src_hbm.at[pl.ds(base, EPW)], srcb)
    pltpu.sync_copy(dst_hbm.at[pl.ds(base, EPW)], dstb)

    def issue_gathers(i, sl):
        sidx = srcb.at[pl.ds(i * CHUNK, CHUNK)]
        didx = dstb.at[pl.ds(i * CHUNK, CHUNK)]
        pltpu.async_copy(svt_hbm.at[sidx], sgv.at[sl], gsem.at[sl])
        pltpu.async_copy(ewt_hbm.at[didx], egv.at[sl], gsem.at[sl])

    def wait_gathers(i, sl):
        sidx = srcb.at[pl.ds(i * CHUNK, CHUNK)]
        didx = dstb.at[pl.ds(i * CHUNK, CHUNK)]
        pltpu.make_async_copy(svt_hbm.at[sidx], sgv.at[sl], gsem.at[sl]).wait()
        pltpu.make_async_copy(ewt_hbm.at[didx], egv.at[sl], gsem.at[sl]).wait()

    def process(i, sl):
        wait_gathers(i, sl)
        cb = base + i * CHUNK
        pltpu.async_copy(sgv.at[sl], svg_out.at[pl.ds(cb, CHUNK)], wsem.at[sl])
        pltpu.async_copy(egv.at[sl], ewg_out.at[pl.ds(cb, CHUNK)], wsem.at[sl])

    def wait_writes(i, sl):
        cb = base + i * CHUNK
        pltpu.make_async_copy(sgv.at[sl], svg_out.at[pl.ds(cb, CHUNK)],
                              wsem.at[sl]).wait()
        pltpu.make_async_copy(egv.at[sl], ewg_out.at[pl.ds(cb, CHUNK)],
                              wsem.at[sl]).wait()

    issue_gathers(0, 0)

    def body(i, carry):
        sl = lax.rem(i, 2)
        pv = lax.rem(i - 1, 2)

        @pl.when(i >= 2)
        def _():
            wait_writes(i - 2, sl)

        issue_gathers(i, sl)
        process(i - 1, pv)
        return carry

    lax.fori_loop(1, NCHUNK, body, 0, unroll=1)
    last = NCHUNK - 1
    process(last, lax.rem(last, 2))
    wait_writes(last - 1, lax.rem(last - 1, 2))
    wait_writes(last, lax.rem(last, 2))


_sc_gather = functools.partial(
    pl.kernel,
    out_type=(
        jax.ShapeDtypeStruct((E, H), jnp.float32),
        jax.ShapeDtypeStruct((E, H), jnp.float32),
    ),
    mesh=plsc.VectorSubcoreMesh(core_axis_name="c", subcore_axis_name="s"),
    scratch_types=[
        pltpu.VMEM((EPW,), jnp.int32),
        pltpu.VMEM((EPW,), jnp.int32),
        pltpu.VMEM((2, CHUNK, H), jnp.float32),
        pltpu.VMEM((2, CHUNK, H), jnp.float32),
        pltpu.SemaphoreType.DMA((2,)),
        pltpu.SemaphoreType.DMA((2,)),
    ],
)(_sc_gather_body)


def _sc_scatter_body(msg_hbm, dst3_hbm, zeros_hbm, agg_out,
                     dst2d, msgv, acc, lsem, ssem):
    c = lax.axis_index("c")
    s = lax.axis_index("s")
    wid = s * NC + c
    base = wid * EPW

    pltpu.sync_copy(dst3_hbm.at[wid], dst2d)

    pltpu.sync_copy(zeros_hbm.at[pl.ds(s * RPS, RPS)], acc.at[pl.ds(s * RPS, RPS)])
    @pl.when(s == 0)
    def _():
        pltpu.sync_copy(zeros_hbm.at[pl.ds(NS * RPS, TAIL)],
                        acc.at[pl.ds(NS * RPS, TAIL)])
    plsc.subcore_barrier()

    def issue_load(i, sl):
        cb = base + i * CHUNK
        pltpu.async_copy(msg_hbm.at[pl.ds(cb, CHUNK)], msgv.at[sl], lsem.at[sl])

    def wait_load(i, sl):
        cb = base + i * CHUNK
        pltpu.make_async_copy(msg_hbm.at[pl.ds(cb, CHUNK)], msgv.at[sl],
                              lsem.at[sl]).wait()

    def issue_scatter(i, sl):
        pltpu.async_copy(msgv.at[sl], acc.at[dst2d.at[i]], ssem.at[sl],
                         add=True)

    def wait_scatter(i, sl):
        pltpu.make_async_copy(msgv.at[sl], acc.at[dst2d.at[i]],
                              ssem.at[sl]).wait()

    issue_load(0, 0)

    def body(i, carry):
        sl = lax.rem(i, 2)
        pv = lax.rem(i - 1, 2)

        @pl.when(i >= 2)
        def _():
            wait_scatter(i - 2, sl)

        issue_load(i, sl)
        wait_load(i - 1, pv)
        issue_scatter(i - 1, pv)
        return carry

    lax.fori_loop(1, NCHUNK, body, 0, unroll=1)
    last = NCHUNK - 1
    wait_load(last, lax.rem(last, 2))
    issue_scatter(last, lax.rem(last, 2))
    wait_scatter(last - 1, lax.rem(last - 1, 2))
    wait_scatter(last, lax.rem(last, 2))

    plsc.subcore_barrier()
    pltpu.sync_copy(acc.at[pl.ds(s * RPS, RPS)],
                    agg_out.at[c, pl.ds(s * RPS, RPS)])
    @pl.when(s == 0)
    def _():
        pltpu.sync_copy(acc.at[pl.ds(NS * RPS, TAIL)],
                        agg_out.at[c, pl.ds(NS * RPS, TAIL)])


_sc_scatter = functools.partial(
    pl.kernel,
    out_type=jax.ShapeDtypeStruct((NC, N, H), jnp.float32),
    mesh=plsc.VectorSubcoreMesh(core_axis_name="c", subcore_axis_name="s"),
    scratch_types=[
        pltpu.VMEM((NCHUNK, CHUNK), jnp.int32),
        pltpu.VMEM((2, CHUNK, H), jnp.float32),
        pltpu.VMEM_SHARED((N, H), jnp.float32),
        pltpu.SemaphoreType.DMA((2,)),
        pltpu.SemaphoreType.DMA((2,)),
    ],
)(_sc_scatter_body)


def kernel(nodes_feature, x, edges_feature, e, mask, t, edge_index, params):
    f32 = jnp.float32
    src = edge_index[0].astype(jnp.int32)
    dst = edge_index[1].astype(jnp.int32)
    dst3 = dst.reshape(NW, NCHUNK, CHUNK)

    x2d = x.astype(jnp.int32).reshape(N, 1)
    e2d = e.astype(jnp.int32).reshape(E, 1)
    m2d = mask.astype(jnp.int32).reshape(E, 1)
    ef2d = edges_feature.reshape(E, 1)

    x_h = _emb_nodes(nodes_feature, x2d, params['node_w'],
                     params['node_b'].reshape(1, H), params['x_embed'])
    etv = _time_vecs(t.reshape(1, 1),
                     params['time_w1'], params['time_b1'].reshape(1, H),
                     params['time_w2'], params['time_b2'].reshape(1, H),
                     jnp.stack([et['w'] for et in params['edge_time']]),
                     jnp.stack([et['b'] for et in params['edge_time']]))

    zeros_nh = jnp.zeros((N, H), f32)
    zeros_1h = jnp.zeros((1, H), f32)

    e_h = None
    e_out = None
    nlayers = [(bi, li) for bi, nl in enumerate(params['blocks'])
               for li in range(len(nl))]
    for bi, li in nlayers:
        p = params['blocks'][bi][li]
        is_first = (bi == 0 and li == 0)
        is_last = (bi, li) == nlayers[-1]
        w4 = jnp.concatenate([p['U'], p['Ew'], p['D'], p['V']], axis=1)
        mf, mb = _node_mats(x_h, w4)
        ux = mf[:, 0:H]
        ewt = mf[:, H:2 * H]
        svt = jax.lax.bitcast_convert_type(mb.reshape(N, H, 2), f32)
        svg, ewg = _sc_gather(svt, ewt, src, dst)
        svb = jax.lax.bitcast_convert_type(svg, jnp.bfloat16).reshape(E, 2 * H)
        tvec = etv[bi:bi + 1] if li == len(params['blocks'][bi]) - 1 \
            else zeros_1h
        g = p['ln_e_g'].reshape(1, H)
        b = p['ln_e_b'].reshape(1, H)
        if is_first:
            msg, e_h = _edge_fused_first(
                ef2d, e2d, m2d, params['edge_w'],
                params['edge_b'].reshape(1, H), params['e_embed'],
                params['mask_embed'], svb, ewg, p['C'], g, b, tvec)
        elif is_last:
            msg, e_out = _edge_fused_last(
                e_h, svb, ewg, p['C'], g, b, tvec,
                params['out_edge_w'], params['out_edge_b'].reshape(1, 2))
        else:
            msg, e_h = _edge_fused_mid(e_h, svb, ewg, p['C'], g, b, tvec)
        agg = _sc_scatter(msg, dst3, zeros_nh)
        x_h = _node_update(x_h, ux, agg,
                           p['ln_x_g'].reshape(1, H),
                           p['ln_x_b'].reshape(1, H))

    x_out = _proj(x_h, params['out_node_w'],
                  params['out_node_b'].reshape(1, 2), N, 2000)
    return (x_out, e_out)


# speedup vs baseline: 1.9874x; 1.9874x over previous
import functools

import jax
import jax.numpy as jnp
from jax import lax
from jax.experimental import pallas as pl
from jax.experimental.pallas import tpu as pltpu
from jax.experimental.pallas import tpu_sc as plsc

N = 10000
E = 320000
H = 128
HALF = H // 2

NC = 2
NS = 16
NW = NC * NS
EPW = E // NW
CHUNK = 80
NCHUNK = EPW // CHUNK
RPS = 624
TAIL = N - NS * RPS

_PREC = jax.lax.Precision.HIGHEST


def _dot(a, b):
    return jnp.dot(a, b, preferred_element_type=jnp.float32, precision=_PREC)


def _emb_node_body(nf_ref, x_ref, nw_ref, nb_ref, xe_ref, out_ref):
    nf = nf_ref[...]
    xv = x_ref[...]
    w = nw_ref[...]
    out_ref[...] = (nf[:, 0:1] * w[0:1, :] + nf[:, 1:2] * w[1:2, :]
                    + nb_ref[...]
                    + jnp.where(xv == 1, xe_ref[1:2, :], xe_ref[0:1, :]))


def _emb_nodes(nodes_feature, x2d, node_w, node_b, x_embed):
    nb = 2000
    grid = N // nb
    return pl.pallas_call(
        _emb_node_body,
        grid=(grid,),
        in_specs=[
            pl.BlockSpec((nb, 2), lambda i: (i, 0)),
            pl.BlockSpec((nb, 1), lambda i: (i, 0)),
            pl.BlockSpec((2, H), lambda i: (0, 0)),
            pl.BlockSpec((1, H), lambda i: (0, 0)),
            pl.BlockSpec((2, H), lambda i: (0, 0)),
        ],
        out_specs=pl.BlockSpec((nb, H), lambda i: (i, 0)),
        out_shape=jax.ShapeDtypeStruct((N, H), jnp.float32),
    )(nodes_feature, x2d, node_w, node_b, x_embed)


def _time_body(t_ref, w1_ref, b1_ref, w2_ref, b2_ref, etw_ref, etb_ref, out_ref):
    idx = jax.lax.broadcasted_iota(jnp.int32, (1, H), 1).astype(jnp.float32)
    k = jnp.where(idx < HALF, idx, idx - HALF)
    freq = jnp.exp((-jnp.log(10000.0) / HALF) * k)
    arg = t_ref[0, 0] * freq
    te = jnp.where(idx < HALF, jnp.sin(arg), jnp.cos(arg))
    h1 = jnp.maximum(_dot(te, w1_ref[...]) + b1_ref[...], 0.0)
    th = _dot(h1, w2_ref[...]) + b2_ref[...]
    rt = jnp.maximum(th, 0.0)
    out_ref[0:1, :] = _dot(rt, etw_ref[0]) + etb_ref[0:1, :]
    out_ref[1:2, :] = _dot(rt, etw_ref[1]) + etb_ref[1:2, :]


def _time_vecs(t11, w1, b1, w2, b2, etw, etb):
    return pl.pallas_call(
        _time_body,
        out_shape=jax.ShapeDtypeStruct((2, H), jnp.float32),
    )(t11, w1, b1, w2, b2, etw, etb)


def _mm_body(a_ref, w_ref, out_ref):
    out_ref[...] = _dot(a_ref[...], w_ref[...])


def _node_mats(x_h, w4):
    nb = 2000
    grid = N // nb
    return pl.pallas_call(
        _mm_body,
        grid=(grid,),
        in_specs=[
            pl.BlockSpec((nb, H), lambda i: (i, 0)),
            pl.BlockSpec((H, 4 * H), lambda i: (0, 0)),
        ],
        out_specs=pl.BlockSpec((nb, 4 * H), lambda i: (i, 0)),
        out_shape=jax.ShapeDtypeStruct((N, 4 * H), jnp.float32),
    )(x_h, w4)


def _edge_core(eh, eg_ref, vxg_ref, c_ref, g_ref, b_ref, tv_ref):
    en = _dot(eh, c_ref[...]) + eg_ref[...]
    gates = 1.0 / (1.0 + jnp.exp(-en))
    msg = gates * vxg_ref[...]
    mu = jnp.mean(en, axis=-1, keepdims=True)
    d = en - mu
    var = jnp.mean(d * d, axis=-1, keepdims=True)
    ln = g_ref[...] * d * jax.lax.rsqrt(var + 1e-5) + b_ref[...]
    ehn = eh + jnp.maximum(ln, 0.0) + tv_ref[...]
    return msg, ehn


def _ef_mid_body(eh_ref, eg_ref, vxg_ref, c_ref, g_ref, b_ref, tv_ref,
                 msg_ref, ehn_ref):
    msg_ref[...], ehn_ref[...] = _edge_core(
        eh_ref[...], eg_ref, vxg_ref, c_ref, g_ref, b_ref, tv_ref)


def _ef_first_body(ef_ref, e_ref, m_ref, ew0_ref, eb0_ref, ee_ref, me_ref,
                   eg_ref, vxg_ref, c_ref, g_ref, b_ref, tv_ref,
                   msg_ref, ehn_ref):
    eh = (ef_ref[...] * ew0_ref[...] + eb0_ref[...]
          + jnp.where(e_ref[...] == 1, ee_ref[1:2, :], ee_ref[0:1, :])
          + jnp.where(m_ref[...] == 1, me_ref[1:2, :], me_ref[0:1, :]))
    msg_ref[...], ehn_ref[...] = _edge_core(
        eh, eg_ref, vxg_ref, c_ref, g_ref, b_ref, tv_ref)


def _ef_last_body(eh_ref, eg_ref, vxg_ref, c_ref, g_ref, b_ref, tv_ref,
                  ow_ref, ob_ref, msg_ref, eo_ref):
    msg, ehn = _edge_core(
        eh_ref[...], eg_ref, vxg_ref, c_ref, g_ref, b_ref, tv_ref)
    msg_ref[...] = msg
    eo_ref[...] = _dot(ehn, ow_ref[...]) + ob_ref[...]


_EB = 4000
_COMMON_SPECS = [
    pl.BlockSpec((_EB, H), lambda i: (i, 0)),
    pl.BlockSpec((_EB, H), lambda i: (i, 0)),
    pl.BlockSpec((H, H), lambda i: (0, 0)),
    pl.BlockSpec((1, H), lambda i: (0, 0)),
    pl.BlockSpec((1, H), lambda i: (0, 0)),
    pl.BlockSpec((1, H), lambda i: (0, 0)),
]
_MSG_EHN_OUT = dict(
    out_specs=[
        pl.BlockSpec((_EB, H), lambda i: (i, 0)),
        pl.BlockSpec((_EB, H), lambda i: (i, 0)),
    ],
    out_shape=[
        jax.ShapeDtypeStruct((E, H), jnp.float32),
        jax.ShapeDtypeStruct((E, H), jnp.float32),
    ],
)


def _edge_fused_mid(e_h, sv, ewg, c, g, b, tvec):
    return pl.pallas_call(
        _ef_mid_body,
        grid=(E // _EB,),
        in_specs=[pl.BlockSpec((_EB, H), lambda i: (i, 0))] + _COMMON_SPECS,
        **_MSG_EHN_OUT,
    )(e_h, sv, ewg, c, g, b, tvec)


def _edge_fused_first(ef2d, e2d, m2d, ew0, eb0, ee, me, sv, ewg, c, g, b,
                      tvec):
    return pl.pallas_call(
        _ef_first_body,
        grid=(E // _EB,),
        in_specs=[
            pl.BlockSpec((_EB, 1), lambda i: (i, 0)),
            pl.BlockSpec((_EB, 1), lambda i: (i, 0)),
            pl.BlockSpec((_EB, 1), lambda i: (i, 0)),
            pl.BlockSpec((1, H), lambda i: (0, 0)),
            pl.BlockSpec((1, H), lambda i: (0, 0)),
            pl.BlockSpec((2, H), lambda i: (0, 0)),
            pl.BlockSpec((2, H), lambda i: (0, 0)),
        ] + _COMMON_SPECS,
        **_MSG_EHN_OUT,
    )(ef2d, e2d, m2d, ew0, eb0, ee, me, sv, ewg, c, g, b, tvec)


def _edge_fused_last(e_h, sv, ewg, c, g, b, tvec, ow, ob):
    return pl.pallas_call(
        _ef_last_body,
        grid=(E // _EB,),
        in_specs=[pl.BlockSpec((_EB, H), lambda i: (i, 0))] + _COMMON_SPECS
        + [
            pl.BlockSpec((H, 2), lambda i: (0, 0)),
            pl.BlockSpec((1, 2), lambda i: (0, 0)),
        ],
        out_specs=[
            pl.BlockSpec((_EB, H), lambda i: (i, 0)),
            pl.BlockSpec((_EB, 2), lambda i: (i, 0)),
        ],
        out_shape=[
            jax.ShapeDtypeStruct((E, H), jnp.float32),
            jax.ShapeDtypeStruct((E, 2), jnp.float32),
        ],
    )(e_h, sv, ewg, c, g, b, tvec, ow, ob)


def _node_upd_body(xh_ref, ux_ref, agg_ref, g_ref, b_ref, out_ref):
    xn = ux_ref[...] + agg_ref[0] + agg_ref[1]
    mu = jnp.mean(xn, axis=-1, keepdims=True)
    d = xn - mu
    var = jnp.mean(d * d, axis=-1, keepdims=True)
    ln = g_ref[...] * d * jax.lax.rsqrt(var + 1e-5) + b_ref[...]
    out_ref[...] = xh_ref[...] + jnp.maximum(ln, 0.0)


def _node_update(x_h, ux, agg, g, b):
    nb = 2000
    grid = N // nb
    return pl.pallas_call(
        _node_upd_body,
        grid=(grid,),
        in_specs=[
            pl.BlockSpec((nb, H), lambda i: (i, 0)),
            pl.BlockSpec((nb, H), lambda i: (i, 0)),
            pl.BlockSpec((NC, nb, H), lambda i: (0, i, 0)),
            pl.BlockSpec((1, H), lambda i: (0, 0)),
            pl.BlockSpec((1, H), lambda i: (0, 0)),
        ],
        out_specs=pl.BlockSpec((nb, H), lambda i: (i, 0)),
        out_shape=jax.ShapeDtypeStruct((N, H), jnp.float32),
    )(x_h, ux, agg, g, b)


def _proj_body(h_ref, w_ref, b_ref, out_ref):
    out_ref[...] = _dot(h_ref[...], w_ref[...]) + b_ref[...]


def _proj(h, w, b, rows, rb):
    grid = rows // rb
    return pl.pallas_call(
        _proj_body,
        grid=(grid,),
        in_specs=[
            pl.BlockSpec((rb, H), lambda i: (i, 0)),
            pl.BlockSpec((H, 2), lambda i: (0, 0)),
            pl.BlockSpec((1, 2), lambda i: (0, 0)),
        ],
        out_specs=pl.BlockSpec((rb, 2), lambda i: (i, 0)),
        out_shape=jax.ShapeDtypeStruct((rows, 2), jnp.float32),
    )(h, w, b)


def _sc_gather_body(dxt_hbm, ewt_hbm, vxt_hbm, src_hbm, dst_hbm,
                    eg_out, vxg_out,
                    srcb, dstb, dgv, egv, vgv, gsem, wsem):
    c = lax.axis_index("c")
    s = lax.axis_index("s")
    wid = s * NC + c
    base = wid * EPW

    pltpu.sync_copy(src_hbm.at[pl.ds(base, EPW)], srcb)
    pltpu.sync_copy(dst_hbm.at[pl.ds(base, EPW)], dstb)

    def issue_gathers(i, sl):
        sidx = srcb.at[pl.ds(i * CHUNK, CHUNK)]
        didx = dstb.at[pl.ds(i * CHUNK, CHUNK)]
        pltpu.async_copy(dxt_hbm.at[sidx], dgv.at[sl], gsem.at[sl])
        pltpu.async_copy(ewt_hbm.at[didx], egv.at[sl], gsem.at[sl])
        pltpu.async_copy(vxt_hbm.at[sidx], vgv.at[sl], gsem.at[sl])

    def wait_gathers(i, sl):
        sidx = srcb.at[pl.ds(i * CHUNK, CHUNK)]
        didx = dstb.at[pl.ds(i * CHUNK, CHUNK)]
        pltpu.make_async_copy(dxt_hbm.at[sidx], dgv.at[sl], gsem.at[sl]).wait()
        pltpu.make_async_copy(ewt_hbm.at[didx], egv.at[sl], gsem.at[sl]).wait()
        pltpu.make_async_copy(vxt_hbm.at[sidx], vgv.at[sl], gsem.at[sl]).wait()

    def process(i, sl):
        wait_gathers(i, sl)

        def add_row(r, carry):
            for k in range(8):
                ix = pl.ds(k * 16, 16)
                egv[sl, r, ix] = egv[sl, r, ix] + dgv[sl, r, ix]
            return carry

        lax.fori_loop(0, CHUNK, add_row, 0, unroll=1)
        cb = base + i * CHUNK
        pltpu.async_copy(egv.at[sl], eg_out.at[pl.ds(cb, CHUNK)], wsem.at[sl])
        pltpu.async_copy(vgv.at[sl], vxg_out.at[pl.ds(cb, CHUNK)], wsem.at[sl])

    def wait_writes(i, sl):
        cb = base + i * CHUNK
        pltpu.make_async_copy(egv.at[sl], eg_out.at[pl.ds(cb, CHUNK)],
                              wsem.at[sl]).wait()
        pltpu.make_async_copy(vgv.at[sl], vxg_out.at[pl.ds(cb, CHUNK)],
                              wsem.at[sl]).wait()

    issue_gathers(0, 0)

    def body(i, carry):
        sl = lax.rem(i, 2)
        pv = lax.rem(i - 1, 2)

        @pl.when(i >= 2)
        def _():
            wait_writes(i - 2, sl)

        issue_gathers(i, sl)
        process(i - 1, pv)
        return carry

    lax.fori_loop(1, NCHUNK, body, 0, unroll=1)
    last = NCHUNK - 1
    process(last, lax.rem(last, 2))
    wait_writes(last - 1, lax.rem(last - 1, 2))
    wait_writes(last, lax.rem(last, 2))


_sc_gather = functools.partial(
    pl.kernel,
    out_type=(
        jax.ShapeDtypeStruct((E, H), jnp.float32),
        jax.ShapeDtypeStruct((E, H), jnp.float32),
    ),
    mesh=plsc.VectorSubcoreMesh(core_axis_name="c", subcore_axis_name="s"),
    scratch_types=[
        pltpu.VMEM((EPW,), jnp.int32),
        pltpu.VMEM((EPW,), jnp.int32),
        pltpu.VMEM((2, CHUNK, H), jnp.float32),
        pltpu.VMEM((2, CHUNK, H), jnp.float32),
        pltpu.VMEM((2, CHUNK, H), jnp.float32),
        pltpu.SemaphoreType.DMA((2,)),
        pltpu.SemaphoreType.DMA((2,)),
    ],
)(_sc_gather_body)


def _sc_scatter_body(msg_hbm, dst3_hbm, zeros_hbm, agg_out,
                     dst2d, msgv, acc, lsem, ssem):
    c = lax.axis_index("c")
    s = lax.axis_index("s")
    wid = s * NC + c
    base = wid * EPW

    pltpu.sync_copy(dst3_hbm.at[wid], dst2d)

    pltpu.sync_copy(zeros_hbm.at[pl.ds(s * RPS, RPS)], acc.at[pl.ds(s * RPS, RPS)])
    @pl.when(s == 0)
    def _():
        pltpu.sync_copy(zeros_hbm.at[pl.ds(NS * RPS, TAIL)],
                        acc.at[pl.ds(NS * RPS, TAIL)])
    plsc.subcore_barrier()

    def issue_load(i, sl):
        cb = base + i * CHUNK
        pltpu.async_copy(msg_hbm.at[pl.ds(cb, CHUNK)], msgv.at[sl], lsem.at[sl])

    def wait_load(i, sl):
        cb = base + i * CHUNK
        pltpu.make_async_copy(msg_hbm.at[pl.ds(cb, CHUNK)], msgv.at[sl],
                              lsem.at[sl]).wait()

    def issue_scatter(i, sl):
        pltpu.async_copy(msgv.at[sl], acc.at[dst2d.at[i]], ssem.at[sl],
                         add=True)

    def wait_scatter(i, sl):
        pltpu.make_async_copy(msgv.at[sl], acc.at[dst2d.at[i]],
                              ssem.at[sl]).wait()

    issue_load(0, 0)

    def body(i, carry):
        sl = lax.rem(i, 2)
        pv = lax.rem(i - 1, 2)

        @pl.when(i >= 2)
        def _():
            wait_scatter(i - 2, sl)

        issue_load(i, sl)
        wait_load(i - 1, pv)
        issue_scatter(i - 1, pv)
        return carry

    lax.fori_loop(1, NCHUNK, body, 0, unroll=1)
    last = NCHUNK - 1
    wait_load(last, lax.rem(last, 2))
    issue_scatter(last, lax.rem(last, 2))
    wait_scatter(last - 1, lax.rem(last - 1, 2))
    wait_scatter(last, lax.rem(last, 2))

    plsc.subcore_barrier()
    pltpu.sync_copy(acc.at[pl.ds(s * RPS, RPS)],
                    agg_out.at[c, pl.ds(s * RPS, RPS)])
    @pl.when(s == 0)
    def _():
        pltpu.sync_copy(acc.at[pl.ds(NS * RPS, TAIL)],
                        agg_out.at[c, pl.ds(NS * RPS, TAIL)])


_sc_scatter = functools.partial(
    pl.kernel,
    out_type=jax.ShapeDtypeStruct((NC, N, H), jnp.float32),
    mesh=plsc.VectorSubcoreMesh(core_axis_name="c", subcore_axis_name="s"),
    scratch_types=[
        pltpu.VMEM((NCHUNK, CHUNK), jnp.int32),
        pltpu.VMEM((2, CHUNK, H), jnp.float32),
        pltpu.VMEM_SHARED((N, H), jnp.float32),
        pltpu.SemaphoreType.DMA((2,)),
        pltpu.SemaphoreType.DMA((2,)),
    ],
)(_sc_scatter_body)


def kernel(nodes_feature, x, edges_feature, e, mask, t, edge_index, params):
    f32 = jnp.float32
    src = edge_index[0].astype(jnp.int32)
    dst = edge_index[1].astype(jnp.int32)
    dst3 = dst.reshape(NW, NCHUNK, CHUNK)

    x2d = x.astype(jnp.int32).reshape(N, 1)
    e2d = e.astype(jnp.int32).reshape(E, 1)
    m2d = mask.astype(jnp.int32).reshape(E, 1)
    ef2d = edges_feature.reshape(E, 1)

    x_h = _emb_nodes(nodes_feature, x2d, params['node_w'],
                     params['node_b'].reshape(1, H), params['x_embed'])
    etv = _time_vecs(t.reshape(1, 1),
                     params['time_w1'], params['time_b1'].reshape(1, H),
                     params['time_w2'], params['time_b2'].reshape(1, H),
                     jnp.stack([et['w'] for et in params['edge_time']]),
                     jnp.stack([et['b'] for et in params['edge_time']]))

    zeros_nh = jnp.zeros((N, H), f32)
    zeros_1h = jnp.zeros((1, H), f32)

    e_h = None
    e_out = None
    nlayers = [(bi, li) for bi, nl in enumerate(params['blocks'])
               for li in range(len(nl))]
    for bi, li in nlayers:
        p = params['blocks'][bi][li]
        is_first = (bi == 0 and li == 0)
        is_last = (bi, li) == nlayers[-1]
        w4 = jnp.concatenate([p['U'], p['V'], p['D'], p['Ew']], axis=1)
        m4 = _node_mats(x_h, w4)
        ux = m4[:, 0:H]
        vxt = m4[:, H:2 * H]
        dxt = m4[:, 2 * H:3 * H]
        ewt = m4[:, 3 * H:4 * H]
        egath, vxg = _sc_gather(dxt, ewt, vxt, src, dst)
        tvec = etv[bi:bi + 1] if li == len(params['blocks'][bi]) - 1 \
            else zeros_1h
        g = p['ln_e_g'].reshape(1, H)
        b = p['ln_e_b'].reshape(1, H)
        if is_first:
            msg, e_h = _edge_fused_first(
                ef2d, e2d, m2d, params['edge_w'],
                params['edge_b'].reshape(1, H), params['e_embed'],
                params['mask_embed'], egath, vxg, p['C'], g, b, tvec)
        elif is_last:
            msg, e_out = _edge_fused_last(
                e_h, egath, vxg, p['C'], g, b, tvec,
                params['out_edge_w'], params['out_edge_b'].reshape(1, 2))
        else:
            msg, e_h = _edge_fused_mid(e_h, egath, vxg, p['C'], g, b, tvec)
        agg = _sc_scatter(msg, dst3, zeros_nh)
        x_h = _node_update(x_h, ux, agg,
                           p['ln_x_g'].reshape(1, H),
                           p['ln_x_b'].reshape(1, H))

    x_out = _proj(x_h, params['out_node_w'],
                  params['out_node_b'].reshape(1, 2), N, 2000)
    return (x_out, e_out)


# speedup vs baseline: 2.5973x; 1.3069x over previous
import functools

import jax
import jax.numpy as jnp
from jax import lax
from jax.experimental import pallas as pl
from jax.experimental.pallas import tpu as pltpu
from jax.experimental.pallas import tpu_sc as plsc

N = 10000
E = 320000
H = 128
HALF = H // 2

NC = 2
NS = 16
NW = NC * NS
EPW = E // NW
CHUNK = 80
NCHUNK = EPW // CHUNK
RPS = 624
TAIL = N - NS * RPS

_PREC = jax.lax.Precision.HIGHEST


def _dot(a, b):
    return jnp.dot(a, b, preferred_element_type=jnp.float32, precision=_PREC)


def _emb_node_body(nf_ref, x_ref, nw_ref, nb_ref, xe_ref, out_ref):
    nf = nf_ref[...]
    xv = x_ref[...]
    w = nw_ref[...]
    out_ref[...] = (nf[:, 0:1] * w[0:1, :] + nf[:, 1:2] * w[1:2, :]
                    + nb_ref[...]
                    + jnp.where(xv == 1, xe_ref[1:2, :], xe_ref[0:1, :]))


def _emb_nodes(nodes_feature, x2d, node_w, node_b, x_embed):
    nb = 2000
    grid = N // nb
    return pl.pallas_call(
        _emb_node_body,
        grid=(grid,),
        in_specs=[
            pl.BlockSpec((nb, 2), lambda i: (i, 0)),
            pl.BlockSpec((nb, 1), lambda i: (i, 0)),
            pl.BlockSpec((2, H), lambda i: (0, 0)),
            pl.BlockSpec((1, H), lambda i: (0, 0)),
            pl.BlockSpec((2, H), lambda i: (0, 0)),
        ],
        out_specs=pl.BlockSpec((nb, H), lambda i: (i, 0)),
        out_shape=jax.ShapeDtypeStruct((N, H), jnp.float32),
    )(nodes_feature, x2d, node_w, node_b, x_embed)


def _time_body(t_ref, w1_ref, b1_ref, w2_ref, b2_ref, etw_ref, etb_ref, out_ref):
    idx = jax.lax.broadcasted_iota(jnp.int32, (1, H), 1).astype(jnp.float32)
    k = jnp.where(idx < HALF, idx, idx - HALF)
    freq = jnp.exp((-jnp.log(10000.0) / HALF) * k)
    arg = t_ref[0, 0] * freq
    te = jnp.where(idx < HALF, jnp.sin(arg), jnp.cos(arg))
    h1 = jnp.maximum(_dot(te, w1_ref[...]) + b1_ref[...], 0.0)
    th = _dot(h1, w2_ref[...]) + b2_ref[...]
    rt = jnp.maximum(th, 0.0)
    out_ref[0:1, :] = _dot(rt, etw_ref[0]) + etb_ref[0:1, :]
    out_ref[1:2, :] = _dot(rt, etw_ref[1]) + etb_ref[1:2, :]


def _time_vecs(t11, w1, b1, w2, b2, etw, etb):
    return pl.pallas_call(
        _time_body,
        out_shape=jax.ShapeDtypeStruct((2, H), jnp.float32),
    )(t11, w1, b1, w2, b2, etw, etb)


def _mm_body(a_ref, w_ref, of_ref, sv_ref):
    prod = _dot(a_ref[...], w_ref[...])
    of_ref[...] = prod[:, 0:2 * H]
    d16 = jax.lax.bitcast_convert_type(
        prod[:, 2 * H:3 * H].astype(jnp.bfloat16), jnp.uint16)
    v16 = jax.lax.bitcast_convert_type(
        prod[:, 3 * H:4 * H].astype(jnp.bfloat16), jnp.uint16)
    w32 = d16.astype(jnp.uint32) | (v16.astype(jnp.uint32) << 16)
    sv_ref[...] = jax.lax.bitcast_convert_type(w32, jnp.float32)


def _node_mats(x_h, w4):
    nb = 2000
    grid = N // nb
    return pl.pallas_call(
        _mm_body,
        grid=(grid,),
        in_specs=[
            pl.BlockSpec((nb, H), lambda i: (i, 0)),
            pl.BlockSpec((H, 4 * H), lambda i: (0, 0)),
        ],
        out_specs=[
            pl.BlockSpec((nb, 2 * H), lambda i: (i, 0)),
            pl.BlockSpec((nb, H), lambda i: (i, 0)),
        ],
        out_shape=[
            jax.ShapeDtypeStruct((N, 2 * H), jnp.float32),
            jax.ShapeDtypeStruct((N, H), jnp.float32),
        ],
    )(x_h, w4)


def _edge_core(eh, sv_ref, ewg_ref, c_ref, g_ref, b_ref, tv_ref):
    w = jax.lax.bitcast_convert_type(sv_ref[...], jnp.int32)
    dx = jax.lax.bitcast_convert_type(w << 16, jnp.float32)
    vx = jax.lax.bitcast_convert_type(w & jnp.int32(-65536), jnp.float32)
    en = _dot(eh, c_ref[...]) + dx + ewg_ref[...]
    gates = 1.0 / (1.0 + jnp.exp(-en))
    msg = gates * vx
    mu = jnp.mean(en, axis=-1, keepdims=True)
    d = en - mu
    var = jnp.mean(d * d, axis=-1, keepdims=True)
    ln = g_ref[...] * d * jax.lax.rsqrt(var + 1e-5) + b_ref[...]
    ehn = eh + jnp.maximum(ln, 0.0) + tv_ref[...]
    return msg, ehn


def _ef_mid_body(eh_ref, eg_ref, vxg_ref, c_ref, g_ref, b_ref, tv_ref,
                 msg_ref, ehn_ref):
    msg_ref[...], ehn_ref[...] = _edge_core(
        eh_ref[...], eg_ref, vxg_ref, c_ref, g_ref, b_ref, tv_ref)


def _ef_first_body(ef_ref, e_ref, m_ref, ew0_ref, eb0_ref, ee_ref, me_ref,
                   eg_ref, vxg_ref, c_ref, g_ref, b_ref, tv_ref,
                   msg_ref, ehn_ref):
    eh = (ef_ref[...] * ew0_ref[...] + eb0_ref[...]
          + jnp.where(e_ref[...] == 1, ee_ref[1:2, :], ee_ref[0:1, :])
          + jnp.where(m_ref[...] == 1, me_ref[1:2, :], me_ref[0:1, :]))
    msg_ref[...], ehn_ref[...] = _edge_core(
        eh, eg_ref, vxg_ref, c_ref, g_ref, b_ref, tv_ref)


def _ef_last_body(eh_ref, eg_ref, vxg_ref, c_ref, g_ref, b_ref, tv_ref,
                  ow_ref, ob_ref, msg_ref, eo_ref):
    msg, ehn = _edge_core(
        eh_ref[...], eg_ref, vxg_ref, c_ref, g_ref, b_ref, tv_ref)
    msg_ref[...] = msg
    eo_ref[...] = _dot(ehn, ow_ref[...]) + ob_ref[...]


_EB = 4000
_COMMON_SPECS = [
    pl.BlockSpec((_EB, H), lambda i: (i, 0)),
    pl.BlockSpec((_EB, H), lambda i: (i, 0)),
    pl.BlockSpec((H, H), lambda i: (0, 0)),
    pl.BlockSpec((1, H), lambda i: (0, 0)),
    pl.BlockSpec((1, H), lambda i: (0, 0)),
    pl.BlockSpec((1, H), lambda i: (0, 0)),
]
_MSG_EHN_OUT = dict(
    out_specs=[
        pl.BlockSpec((_EB, H), lambda i: (i, 0)),
        pl.BlockSpec((_EB, H), lambda i: (i, 0)),
    ],
    out_shape=[
        jax.ShapeDtypeStruct((E, H), jnp.float32),
        jax.ShapeDtypeStruct((E, H), jnp.float32),
    ],
)


def _edge_fused_mid(e_h, sv, ewg, c, g, b, tvec):
    return pl.pallas_call(
        _ef_mid_body,
        grid=(E // _EB,),
        in_specs=[pl.BlockSpec((_EB, H), lambda i: (i, 0))] + _COMMON_SPECS,
        **_MSG_EHN_OUT,
    )(e_h, sv, ewg, c, g, b, tvec)


def _edge_fused_first(ef2d, e2d, m2d, ew0, eb0, ee, me, sv, ewg, c, g, b,
                      tvec):
    return pl.pallas_call(
        _ef_first_body,
        grid=(E // _EB,),
        in_specs=[
            pl.BlockSpec((_EB, 1), lambda i: (i, 0)),
            pl.BlockSpec((_EB, 1), lambda i: (i, 0)),
            pl.BlockSpec((_EB, 1), lambda i: (i, 0)),
            pl.BlockSpec((1, H), lambda i: (0, 0)),
            pl.BlockSpec((1, H), lambda i: (0, 0)),
            pl.BlockSpec((2, H), lambda i: (0, 0)),
            pl.BlockSpec((2, H), lambda i: (0, 0)),
        ] + _COMMON_SPECS,
        **_MSG_EHN_OUT,
    )(ef2d, e2d, m2d, ew0, eb0, ee, me, sv, ewg, c, g, b, tvec)


def _edge_fused_last(e_h, sv, ewg, c, g, b, tvec, ow, ob):
    return pl.pallas_call(
        _ef_last_body,
        grid=(E // _EB,),
        in_specs=[pl.BlockSpec((_EB, H), lambda i: (i, 0))] + _COMMON_SPECS
        + [
            pl.BlockSpec((H, 2), lambda i: (0, 0)),
            pl.BlockSpec((1, 2), lambda i: (0, 0)),
        ],
        out_specs=[
            pl.BlockSpec((_EB, H), lambda i: (i, 0)),
            pl.BlockSpec((_EB, 2), lambda i: (i, 0)),
        ],
        out_shape=[
            jax.ShapeDtypeStruct((E, H), jnp.float32),
            jax.ShapeDtypeStruct((E, 2), jnp.float32),
        ],
    )(e_h, sv, ewg, c, g, b, tvec, ow, ob)


def _node_upd_body(xh_ref, ux_ref, agg_ref, g_ref, b_ref, out_ref):
    xn = ux_ref[...] + agg_ref[0] + agg_ref[1]
    mu = jnp.mean(xn, axis=-1, keepdims=True)
    d = xn - mu
    var = jnp.mean(d * d, axis=-1, keepdims=True)
    ln = g_ref[...] * d * jax.lax.rsqrt(var + 1e-5) + b_ref[...]
    out_ref[...] = xh_ref[...] + jnp.maximum(ln, 0.0)


def _node_update(x_h, ux, agg, g, b):
    nb = 2000
    grid = N // nb
    return pl.pallas_call(
        _node_upd_body,
        grid=(grid,),
        in_specs=[
            pl.BlockSpec((nb, H), lambda i: (i, 0)),
            pl.BlockSpec((nb, H), lambda i: (i, 0)),
            pl.BlockSpec((NC, nb, H), lambda i: (0, i, 0)),
            pl.BlockSpec((1, H), lambda i: (0, 0)),
            pl.BlockSpec((1, H), lambda i: (0, 0)),
        ],
        out_specs=pl.BlockSpec((nb, H), lambda i: (i, 0)),
        out_shape=jax.ShapeDtypeStruct((N, H), jnp.float32),
    )(x_h, ux, agg, g, b)


def _proj_body(h_ref, w_ref, b_ref, out_ref):
    out_ref[...] = _dot(h_ref[...], w_ref[...]) + b_ref[...]


def _proj(h, w, b, rows, rb):
    grid = rows // rb
    return pl.pallas_call(
        _proj_body,
        grid=(grid,),
        in_specs=[
            pl.BlockSpec((rb, H), lambda i: (i, 0)),
            pl.BlockSpec((H, 2), lambda i: (0, 0)),
            pl.BlockSpec((1, 2), lambda i: (0, 0)),
        ],
        out_specs=pl.BlockSpec((rb, 2), lambda i: (i, 0)),
        out_shape=jax.ShapeDtypeStruct((rows, 2), jnp.float32),
    )(h, w, b)


def _sc_gather_body(svt_hbm, ewt_hbm, src_hbm, dst_hbm,
                    svg_out, ewg_out,
                    srcb, dstb, sgv, egv, tbl, isem, gsem, wsem):
    c = lax.axis_index("c")
    s = lax.axis_index("s")
    wid = s * NC + c
    base = wid * EPW

    pltpu.sync_copy(svt_hbm.at[pl.ds(s * RPS, RPS)], tbl.at[pl.ds(s * RPS, RPS)])
    @pl.when(s == 0)
    def _():
        pltpu.sync_copy(svt_hbm.at[pl.ds(NS * RPS, TAIL)],
                        tbl.at[pl.ds(NS * RPS, TAIL)])
    plsc.subcore_barrier()

    def issue_idx(i, sl):
        cb = base + i * CHUNK
        pltpu.async_copy(src_hbm.at[pl.ds(cb, CHUNK)], srcb.at[sl], isem.at[sl])
        pltpu.async_copy(dst_hbm.at[pl.ds(cb, CHUNK)], dstb.at[sl], isem.at[sl])

    def wait_idx(i, sl):
        cb = base + i * CHUNK
        pltpu.make_async_copy(src_hbm.at[pl.ds(cb, CHUNK)], srcb.at[sl],
                              isem.at[sl]).wait()
        pltpu.make_async_copy(dst_hbm.at[pl.ds(cb, CHUNK)], dstb.at[sl],
                              isem.at[sl]).wait()

    def issue_gathers(sl):
        pltpu.async_copy(svt_hbm.at[srcb.at[sl]], sgv.at[sl], gsem.at[sl])
        pltpu.async_copy(ewt_hbm.at[dstb.at[sl]], egv.at[sl], gsem.at[sl])

    def wait_gathers(sl):
        pltpu.make_async_copy(svt_hbm.at[srcb.at[sl]], sgv.at[sl],
                              gsem.at[sl]).wait()
        pltpu.make_async_copy(ewt_hbm.at[dstb.at[sl]], egv.at[sl],
                              gsem.at[sl]).wait()

    def issue_writes(i, sl):
        cb = base + i * CHUNK
        pltpu.async_copy(sgv.at[sl], svg_out.at[pl.ds(cb, CHUNK)], wsem.at[sl])
        pltpu.async_copy(egv.at[sl], ewg_out.at[pl.ds(cb, CHUNK)], wsem.at[sl])

    def wait_writes(i, sl):
        cb = base + i * CHUNK
        pltpu.make_async_copy(sgv.at[sl], svg_out.at[pl.ds(cb, CHUNK)],
                              wsem.at[sl]).wait()
        pltpu.make_async_copy(egv.at[sl], ewg_out.at[pl.ds(cb, CHUNK)],
                              wsem.at[sl]).wait()

    issue_idx(0, 0)

    def body(i, carry):
        sl = lax.rem(i, 2)
        ot = lax.rem(i + 1, 2)
        wait_idx(i, sl)

        @pl.when(i >= 2)
        def _():
            wait_writes(i - 2, sl)

        issue_gathers(sl)

        @pl.when(i >= 1)
        def _():
            wait_gathers(ot)
            issue_writes(i - 1, ot)

        @pl.when(i + 1 < NCHUNK)
        def _():
            issue_idx(i + 1, ot)

        return carry

    lax.fori_loop(0, NCHUNK, body, 0, unroll=1)
    last = NCHUNK - 1
    lsl = lax.rem(last, 2)
    wait_gathers(lsl)
    issue_writes(last, lsl)
    wait_writes(last - 1, lax.rem(last - 1, 2))
    wait_writes(last, lsl)


_sc_gather = functools.partial(
    pl.kernel,
    out_type=(
        jax.ShapeDtypeStruct((E, H), jnp.float32),
        jax.ShapeDtypeStruct((E, H), jnp.float32),
    ),
    mesh=plsc.VectorSubcoreMesh(core_axis_name="c", subcore_axis_name="s"),
    scratch_types=[
        pltpu.VMEM((2, CHUNK), jnp.int32),
        pltpu.VMEM((2, CHUNK), jnp.int32),
        pltpu.VMEM((2, CHUNK, H), jnp.float32),
        pltpu.VMEM((2, CHUNK, H), jnp.float32),
        pltpu.VMEM_SHARED((N, H), jnp.float32),
        pltpu.SemaphoreType.DMA((2,)),
        pltpu.SemaphoreType.DMA((2,)),
        pltpu.SemaphoreType.DMA((2,)),
    ],
)(_sc_gather_body)


def _sc_scatter_body(msg_hbm, dst3_hbm, zeros_hbm, agg_out,
                     dst2d, msgv, acc, lsem, ssem):
    c = lax.axis_index("c")
    s = lax.axis_index("s")
    wid = s * NC + c
    base = wid * EPW

    pltpu.sync_copy(dst3_hbm.at[wid], dst2d)

    pltpu.sync_copy(zeros_hbm.at[pl.ds(s * RPS, RPS)], acc.at[pl.ds(s * RPS, RPS)])
    @pl.when(s == 0)
    def _():
        pltpu.sync_copy(zeros_hbm.at[pl.ds(NS * RPS, TAIL)],
                        acc.at[pl.ds(NS * RPS, TAIL)])
    plsc.subcore_barrier()

    def issue_load(i, sl):
        cb = base + i * CHUNK
        pltpu.async_copy(msg_hbm.at[pl.ds(cb, CHUNK)], msgv.at[sl], lsem.at[sl])

    def wait_load(i, sl):
        cb = base + i * CHUNK
        pltpu.make_async_copy(msg_hbm.at[pl.ds(cb, CHUNK)], msgv.at[sl],
                              lsem.at[sl]).wait()

    def issue_scatter(i, sl):
        pltpu.async_copy(msgv.at[sl], acc.at[dst2d.at[i]], ssem.at[sl],
                         add=True)

    def wait_scatter(i, sl):
        pltpu.make_async_copy(msgv.at[sl], acc.at[dst2d.at[i]],
                              ssem.at[sl]).wait()

    issue_load(0, 0)

    def body(i, carry):
        sl = lax.rem(i, 2)
        pv = lax.rem(i - 1, 2)

        @pl.when(i >= 2)
        def _():
            wait_scatter(i - 2, sl)

        issue_load(i, sl)
        wait_load(i - 1, pv)
        issue_scatter(i - 1, pv)
        return carry

    lax.fori_loop(1, NCHUNK, body, 0, unroll=1)
    last = NCHUNK - 1
    wait_load(last, lax.rem(last, 2))
    issue_scatter(last, lax.rem(last, 2))
    wait_scatter(last - 1, lax.rem(last - 1, 2))
    wait_scatter(last, lax.rem(last, 2))

    plsc.subcore_barrier()
    pltpu.sync_copy(acc.at[pl.ds(s * RPS, RPS)],
                    agg_out.at[c, pl.ds(s * RPS, RPS)])
    @pl.when(s == 0)
    def _():
        pltpu.sync_copy(acc.at[pl.ds(NS * RPS, TAIL)],
                        agg_out.at[c, pl.ds(NS * RPS, TAIL)])


_sc_scatter = functools.partial(
    pl.kernel,
    out_type=jax.ShapeDtypeStruct((NC, N, H), jnp.float32),
    mesh=plsc.VectorSubcoreMesh(core_axis_name="c", subcore_axis_name="s"),
    scratch_types=[
        pltpu.VMEM((NCHUNK, CHUNK), jnp.int32),
        pltpu.VMEM((2, CHUNK, H), jnp.float32),
        pltpu.VMEM_SHARED((N, H), jnp.float32),
        pltpu.SemaphoreType.DMA((2,)),
        pltpu.SemaphoreType.DMA((2,)),
    ],
)(_sc_scatter_body)


def kernel(nodes_feature, x, edges_feature, e, mask, t, edge_index, params):
    f32 = jnp.float32
    src = edge_index[0].astype(jnp.int32)
    dst = edge_index[1].astype(jnp.int32)
    dst3 = dst.reshape(NW, NCHUNK, CHUNK)

    x2d = x.astype(jnp.int32).reshape(N, 1)
    e2d = e.astype(jnp.int32).reshape(E, 1)
    m2d = mask.astype(jnp.int32).reshape(E, 1)
    ef2d = edges_feature.reshape(E, 1)

    x_h = _emb_nodes(nodes_feature, x2d, params['node_w'],
                     params['node_b'].reshape(1, H), params['x_embed'])
    etv = _time_vecs(t.reshape(1, 1),
                     params['time_w1'], params['time_b1'].reshape(1, H),
                     params['time_w2'], params['time_b2'].reshape(1, H),
                     jnp.stack([et['w'] for et in params['edge_time']]),
                     jnp.stack([et['b'] for et in params['edge_time']]))

    zeros_nh = jnp.zeros((N, H), f32)
    zeros_1h = jnp.zeros((1, H), f32)

    e_h = None
    e_out = None
    nlayers = [(bi, li) for bi, nl in enumerate(params['blocks'])
               for li in range(len(nl))]
    for bi, li in nlayers:
        p = params['blocks'][bi][li]
        is_first = (bi == 0 and li == 0)
        is_last = (bi, li) == nlayers[-1]
        w4 = jnp.concatenate([p['U'], p['Ew'], p['D'], p['V']], axis=1)
        mf, svt = _node_mats(x_h, w4)
        ux = mf[:, 0:H]
        ewt = mf[:, H:2 * H]
        svg, ewg = _sc_gather(svt, ewt, src, dst)
        tvec = etv[bi:bi + 1] if li == len(params['blocks'][bi]) - 1 \
            else zeros_1h
        g = p['ln_e_g'].reshape(1, H)
        b = p['ln_e_b'].reshape(1, H)
        if is_first:
            msg, e_h = _edge_fused_first(
                ef2d, e2d, m2d, params['edge_w'],
                params['edge_b'].reshape(1, H), params['e_embed'],
                params['mask_embed'], svg, ewg, p['C'], g, b, tvec)
        elif is_last:
            msg, e_out = _edge_fused_last(
                e_h, svg, ewg, p['C'], g, b, tvec,
                params['out_edge_w'], params['out_edge_b'].reshape(1, 2))
        else:
            msg, e_h = _edge_fused_mid(e_h, svg, ewg, p['C'], g, b, tvec)
        agg = _sc_scatter(msg, dst3, zeros_nh)
        x_h = _node_update(x_h, ux, agg,
                           p['ln_x_g'].reshape(1, H),
                           p['ln_x_b'].reshape(1, H))

    x_out = _proj(x_h, params['out_node_w'],
                  params['out_node_b'].reshape(1, 2), N, 2000)
    return (x_out, e_out)
